# baseline (speedup 1.0000x reference)
import jax
import jax.numpy as jnp
from jax.experimental import pallas as pl


_BN = 512
_W = 4096


def _vq_argmin_kernel(x_ref, xn_ref, cb_ref, cn_ref, io_ref, out_ref):
    xb = x_ref[...].astype(jnp.bfloat16)
    cbb = cb_ref[...].astype(jnp.bfloat16)
    m = jax.lax.dot_general(
        xb, cbb, (((1,), (1,)), ((), ())), preferred_element_type=jnp.float32
    )
    d = (xn_ref[...] + cn_ref[...]) - 2.0 * m
    bn, k = d.shape

    blk0 = jax.lax.slice(d, (0, 0), (bn, _W))
    blk1 = jax.lax.slice(d, (0, _W), (bn, 2 * _W))
    vmin0 = jnp.min(blk0, axis=1, keepdims=True)
    vmin1 = jnp.min(blk1, axis=1, keepdims=True)
    carry0 = vmin0.astype(jnp.bfloat16).astype(jnp.float32)
    take1 = vmin1 < carry0

    dsel = jnp.where(take1, blk1, blk0)
    vsel = jnp.where(take1, vmin1, vmin0)
    mask = jnp.where(dsel == vsel, 1.0, 0.0).astype(jnp.bfloat16)
    r = jax.lax.dot_general(
        mask, io_ref[...], (((1,), (0,)), ((), ())),
        preferred_element_type=jnp.float32)
    idx_fast = (r[:, 0:1] + r[:, 1:2]).astype(jnp.int32)
    has_tie = jnp.any(r[:, 2:3] > 1.5)

    def _slow_scan():
        iota = jax.lax.broadcasted_iota(jnp.int32, (bn, _W), 1)
        return jnp.min(jnp.where(dsel == vsel, iota, k),
                       axis=1, keepdims=True)

    imin = jax.lax.cond(has_tie, _slow_scan, lambda: idx_fast)
    out_ref[...] = imin + jnp.where(take1, _W, 0)


def kernel(x, codebook):
    B = x.shape[0]
    code_dim = codebook.shape[1]
    K = codebook.shape[0]
    flattened = x.reshape(-1, code_dim)
    N = flattened.shape[0]
    xnorm = jnp.sum(flattened ** 2, axis=1, keepdims=True)
    cnorm = jnp.sum(codebook ** 2, axis=1)[None, :]
    ii = jax.lax.iota(jnp.int32, _W)
    io = jnp.stack(
        [((ii // 64) * 64).astype(jnp.float32),
         (ii % 64).astype(jnp.float32),
         jnp.ones((_W,), jnp.float32)], axis=1).astype(jnp.bfloat16)

    codes = pl.pallas_call(
        _vq_argmin_kernel,
        grid=(N // _BN,),
        in_specs=[
            pl.BlockSpec((_BN, code_dim), lambda i: (i, 0)),
            pl.BlockSpec((_BN, 1), lambda i: (i, 0)),
            pl.BlockSpec((K, code_dim), lambda i: (0, 0)),
            pl.BlockSpec((1, K), lambda i: (0, 0)),
            pl.BlockSpec((_W, 3), lambda i: (0, 0)),
        ],
        out_specs=pl.BlockSpec((_BN, 1), lambda i: (i, 0)),
        out_shape=jax.ShapeDtypeStruct((N, 1), jnp.int32),
    )(flattened, xnorm, codebook, cnorm, io)
    return codes.reshape(B, -1)


# speedup vs baseline: 1.0516x; 1.0516x over previous
import jax
import jax.numpy as jnp
from jax.experimental import pallas as pl


_BN = 128
_W = 4096


def _vq_argmin_kernel(x_ref, xn_ref, cb_ref, cn_ref, io_ref, out_ref):
    xb = x_ref[...].astype(jnp.bfloat16)
    cbb = cb_ref[...].astype(jnp.bfloat16)
    m = jax.lax.dot_general(
        xb, cbb, (((1,), (1,)), ((), ())), preferred_element_type=jnp.float32
    )
    d = (xn_ref[...] + cn_ref[...]) - 2.0 * m
    bn, k = d.shape

    blk0 = jax.lax.slice(d, (0, 0), (bn, _W))
    blk1 = jax.lax.slice(d, (0, _W), (bn, 2 * _W))
    vmin0 = jnp.min(blk0, axis=1, keepdims=True)
    vmin1 = jnp.min(blk1, axis=1, keepdims=True)
    carry0 = vmin0.astype(jnp.bfloat16).astype(jnp.float32)
    take1 = vmin1 < carry0

    dsel = jnp.where(take1, blk1, blk0)
    vsel = jnp.where(take1, vmin1, vmin0)
    mask = jnp.where(dsel == vsel, 1.0, 0.0).astype(jnp.bfloat16)
    r = jax.lax.dot_general(
        mask, io_ref[...], (((1,), (0,)), ((), ())),
        preferred_element_type=jnp.float32)
    idx_fast = (r[:, 0:1] + r[:, 1:2]).astype(jnp.int32)
    has_tie = jnp.any(r[:, 2:3] > 1.5)

    def _slow_scan():
        iota = jax.lax.broadcasted_iota(jnp.int32, (bn, _W), 1)
        return jnp.min(jnp.where(dsel == vsel, iota, k),
                       axis=1, keepdims=True)

    imin = jax.lax.cond(has_tie, _slow_scan, lambda: idx_fast)
    out_ref[...] = imin + jnp.where(take1, _W, 0)


def kernel(x, codebook):
    B = x.shape[0]
    code_dim = codebook.shape[1]
    K = codebook.shape[0]
    flattened = x.reshape(-1, code_dim)
    N = flattened.shape[0]
    xnorm = jnp.sum(flattened ** 2, axis=1, keepdims=True)
    cnorm = jnp.sum(codebook ** 2, axis=1)[None, :]
    ii = jax.lax.iota(jnp.int32, _W)
    io = jnp.stack(
        [((ii // 64) * 64).astype(jnp.float32),
         (ii % 64).astype(jnp.float32),
         jnp.ones((_W,), jnp.float32)], axis=1).astype(jnp.bfloat16)

    codes = pl.pallas_call(
        _vq_argmin_kernel,
        grid=(N // _BN,),
        in_specs=[
            pl.BlockSpec((_BN, code_dim), lambda i: (i, 0)),
            pl.BlockSpec((_BN, 1), lambda i: (i, 0)),
            pl.BlockSpec((K, code_dim), lambda i: (0, 0)),
            pl.BlockSpec((1, K), lambda i: (0, 0)),
            pl.BlockSpec((_W, 3), lambda i: (0, 0)),
        ],
        out_specs=pl.BlockSpec((_BN, 1), lambda i: (i, 0)),
        out_shape=jax.ShapeDtypeStruct((N, 1), jnp.int32),
    )(flattened, xnorm, codebook, cnorm, io)
    return codes.reshape(B, -1)


# speedup vs baseline: 1.3003x; 1.2365x over previous
import jax
import jax.numpy as jnp
from jax.experimental import pallas as pl


_BN = 256
_W = 4096


def _vq_argmin_kernel(x_ref, xn_ref, cb_ref, cn_ref, io_ref, out_ref):
    xb = x_ref[...].astype(jnp.bfloat16)
    cbb = cb_ref[...].astype(jnp.bfloat16)
    m = jax.lax.dot_general(
        xb, cbb, (((1,), (1,)), ((), ())), preferred_element_type=jnp.float32
    )
    d = (xn_ref[...] + cn_ref[...]) - 2.0 * m
    bn, k = d.shape

    blk0 = jax.lax.slice(d, (0, 0), (bn, _W))
    blk1 = jax.lax.slice(d, (0, _W), (bn, 2 * _W))
    vmin0 = jnp.min(blk0, axis=1, keepdims=True)
    vmin1 = jnp.min(blk1, axis=1, keepdims=True)
    carry0 = vmin0.astype(jnp.bfloat16).astype(jnp.float32)
    take1 = vmin1 < carry0

    dsel = jnp.where(take1, blk1, blk0)
    vsel = jnp.where(take1, vmin1, vmin0)
    mask = (dsel == vsel).astype(jnp.bfloat16)
    r = jax.lax.dot_general(
        mask, io_ref[...], (((1,), (0,)), ((), ())),
        preferred_element_type=jnp.float32)
    idx_fast = (r[:, 0:1] + r[:, 1:2]).astype(jnp.int32)
    has_tie = jnp.any(r[:, 2:3] > 1.5)

    def _slow_scan():
        iota = jax.lax.broadcasted_iota(jnp.int32, (bn, _W), 1)
        return jnp.min(jnp.where(dsel == vsel, iota, k),
                       axis=1, keepdims=True)

    imin = jax.lax.cond(has_tie, _slow_scan, lambda: idx_fast)
    out_ref[...] = imin + jnp.where(take1, _W, 0)


def kernel(x, codebook):
    B = x.shape[0]
    code_dim = codebook.shape[1]
    K = codebook.shape[0]
    flattened = x.reshape(-1, code_dim)
    N = flattened.shape[0]
    xnorm = jnp.sum(flattened ** 2, axis=1, keepdims=True)
    cnorm = jnp.sum(codebook ** 2, axis=1)[None, :]
    ii = jax.lax.iota(jnp.int32, _W)
    io = jnp.stack(
        [((ii // 64) * 64).astype(jnp.float32),
         (ii % 64).astype(jnp.float32),
         jnp.ones((_W,), jnp.float32)], axis=1).astype(jnp.bfloat16)

    codes = pl.pallas_call(
        _vq_argmin_kernel,
        grid=(N // _BN,),
        in_specs=[
            pl.BlockSpec((_BN, code_dim), lambda i: (i, 0)),
            pl.BlockSpec((_BN, 1), lambda i: (i, 0)),
            pl.BlockSpec((K, code_dim), lambda i: (0, 0)),
            pl.BlockSpec((1, K), lambda i: (0, 0)),
            pl.BlockSpec((_W, 3), lambda i: (0, 0)),
        ],
        out_specs=pl.BlockSpec((_BN, 1), lambda i: (i, 0)),
        out_shape=jax.ShapeDtypeStruct((N, 1), jnp.int32),
    )(flattened, xnorm, codebook, cnorm, io)
    return codes.reshape(B, -1)
